# Initial kernel scaffold; baseline (speedup 1.0000x reference)
import jax
import jax.numpy as jnp
from jax.experimental import pallas as pl


def kernel(x, ln1_w, Wq, Wk, Wv, Wo, ln2_w, Wr, W1, W3, W2):
    raise NotImplementedError("write your pallas kernel here")


# speedup vs baseline: 1.0758x; 1.0758x over previous
import functools
import math

import jax
import jax.numpy as jnp
from jax.experimental import pallas as pl

EPS = 1e-5
N_HEADS = 16
TOP_K = 2
CAP_FACTOR = 1.25
NEG_INF = -1e30


def _k1_body(x_ref, w_ref, wq_ref, wk_ref, wv_ref, q_ref, k_ref, v_ref):
    x = x_ref[...]
    var = jnp.mean(x * x, axis=-1, keepdims=True)
    xln = (x * jax.lax.rsqrt(var + EPS) * w_ref[...]).astype(jnp.bfloat16)
    for wr, outr in ((wq_ref, q_ref), (wk_ref, k_ref), (wv_ref, v_ref)):
        outr[...] = jnp.dot(
            xln, wr[...], preferred_element_type=jnp.float32
        ).astype(jnp.bfloat16)


def _qkv(x, ln1_w, wq16, wk16, wv16, BS):
    S, D = x.shape
    grid = (S // BS,)
    row = pl.BlockSpec((BS, D), lambda i: (i, 0))
    full = pl.BlockSpec((D, D), lambda i: (0, 0))
    wspec = pl.BlockSpec((1, D), lambda i: (0, 0))
    out = jax.ShapeDtypeStruct((S, D), jnp.bfloat16)
    return pl.pallas_call(
        _k1_body,
        grid=grid,
        in_specs=[row, wspec, full, full, full],
        out_specs=(row, row, row),
        out_shape=(out, out, out),
    )(x, ln1_w.reshape(1, D), wq16, wk16, wv16)


def _k2_body(q_ref, k_ref, v_ref, o_ref, *, BQ, S, scale):
    q = q_ref[0]
    k = k_ref[0]
    v = v_ref[0]
    s = jax.lax.dot_general(
        q, k, (((1,), (1,)), ((), ())), preferred_element_type=jnp.float32
    ) * scale
    qb = pl.program_id(1)
    row = qb * BQ + jax.lax.broadcasted_iota(jnp.int32, (BQ, S), 0)
    col = jax.lax.broadcasted_iota(jnp.int32, (BQ, S), 1)
    s = jnp.where(col <= row, s, NEG_INF)
    m = jnp.max(s, axis=-1, keepdims=True)
    e = jnp.exp(s - m)
    p = (e / jnp.sum(e, axis=-1, keepdims=True)).astype(jnp.bfloat16)
    o_ref[0] = jnp.dot(p, v, preferred_element_type=jnp.float32).astype(
        jnp.bfloat16
    )


def _attention(qh, kh, vh, BQ):
    H, S, HD = qh.shape
    body = functools.partial(_k2_body, BQ=BQ, S=S, scale=1.0 / math.sqrt(HD))
    return pl.pallas_call(
        body,
        grid=(H, S // BQ),
        in_specs=[
            pl.BlockSpec((1, BQ, HD), lambda h, i: (h, i, 0)),
            pl.BlockSpec((1, S, HD), lambda h, i: (h, 0, 0)),
            pl.BlockSpec((1, S, HD), lambda h, i: (h, 0, 0)),
        ],
        out_specs=pl.BlockSpec((1, BQ, HD), lambda h, i: (h, i, 0)),
        out_shape=jax.ShapeDtypeStruct((H, S, HD), jnp.bfloat16),
    )(qh, kh, vh)


def _k3_body(ao_ref, wo_ref, x_ref, w2_ref, wr_ref, h_ref, hln_ref, lg_ref):
    att = jnp.dot(ao_ref[...], wo_ref[...], preferred_element_type=jnp.float32)
    h = att + x_ref[...]
    h_ref[...] = h
    var = jnp.mean(h * h, axis=-1, keepdims=True)
    hln = h * jax.lax.rsqrt(var + EPS) * w2_ref[...]
    hln_ref[...] = hln.astype(jnp.bfloat16)
    lg_ref[...] = jnp.dot(
        hln,
        wr_ref[...],
        preferred_element_type=jnp.float32,
        precision=jax.lax.Precision.HIGHEST,
    )


def _post_attn(ao, wo16, x, ln2_w, wr, BS):
    S, D = x.shape
    E = wr.shape[1]
    row = pl.BlockSpec((BS, D), lambda i: (i, 0))
    return pl.pallas_call(
        _k3_body,
        grid=(S // BS,),
        in_specs=[
            row,
            pl.BlockSpec((D, D), lambda i: (0, 0)),
            row,
            pl.BlockSpec((1, D), lambda i: (0, 0)),
            pl.BlockSpec((D, E), lambda i: (0, 0)),
        ],
        out_specs=(row, row, pl.BlockSpec((BS, E), lambda i: (i, 0))),
        out_shape=(
            jax.ShapeDtypeStruct((S, D), jnp.float32),
            jax.ShapeDtypeStruct((S, D), jnp.bfloat16),
            jax.ShapeDtypeStruct((S, E), jnp.float32),
        ),
    )(ao, wo16, x, ln2_w.reshape(1, D), wr)


def _k4_body(lg_ref, st_ref, cidx_ref, gate_ref, *, S, E, C, CH):
    lg = lg_ref[...]
    m = jnp.max(lg, axis=-1, keepdims=True)
    ex = jnp.exp(lg - m)
    probs = ex / jnp.sum(ex, axis=-1, keepdims=True)

    iota_e = jax.lax.broadcasted_iota(jnp.int32, (S, E), 1)
    m1 = jnp.max(probs, axis=-1, keepdims=True)
    i1 = jnp.min(jnp.where(probs == m1, iota_e, E), axis=-1, keepdims=True)
    probs2 = jnp.where(iota_e == i1, -1.0, probs)
    m2 = jnp.max(probs2, axis=-1, keepdims=True)
    i2 = jnp.min(jnp.where(probs2 == m2, iota_e, E), axis=-1, keepdims=True)
    den = m1 + m2 + 1e-9
    g1 = m1 / den
    g2 = m2 / den

    oh0 = (iota_e == i1).astype(jnp.float32)
    oh1 = (iota_e == i2).astype(jnp.float32)
    A = jnp.concatenate([oh0, oh1], axis=0)

    tri = (
        jax.lax.broadcasted_iota(jnp.int32, (CH, CH), 0)
        > jax.lax.broadcasted_iota(jnp.int32, (CH, CH), 1)
    ).astype(jnp.float32)
    carry = jnp.zeros((1, E), jnp.float32)
    prefs = []
    for j in range(2 * S // CH):
        blk = jax.lax.slice(A, (j * CH, 0), ((j + 1) * CH, E))
        prefs.append(
            jnp.dot(
                tri,
                blk,
                preferred_element_type=jnp.float32,
                precision=jax.lax.Precision.HIGHEST,
            )
            + carry
        )
        carry = carry + jnp.sum(blk, axis=0, keepdims=True)
    pos = jnp.concatenate(prefs, axis=0)
    pos_sel = jnp.floor(
        jnp.sum(pos * A, axis=-1, keepdims=True) + 0.5
    )
    valid = (pos_sel < C).astype(jnp.float32)
    pos_c = jnp.clip(pos_sel, 0.0, C - 1.0)

    e_all = jnp.concatenate([i1, i2], axis=0).astype(jnp.float32)
    gate_all = jnp.concatenate([g1, g2], axis=0) * valid
    cidx_ref[...] = (e_all * C + pos_c).astype(jnp.int32)
    gate_ref[...] = gate_all

    iota_c = jax.lax.broadcasted_iota(jnp.int32, (2 * S, C), 1)
    P = (pos_c.astype(jnp.int32) == iota_c).astype(jnp.float32) * valid
    t_all = jax.lax.broadcasted_iota(jnp.int32, (2 * S, 1), 0).astype(
        jnp.float32
    )
    t_all = jnp.where(t_all < S, t_all, t_all - S)
    lhs = A * ((t_all + 1.0) * valid)
    st = jax.lax.dot_general(
        P,
        lhs,
        (((0,), (0,)), ((), ())),
        preferred_element_type=jnp.float32,
        precision=jax.lax.Precision.HIGHEST,
    )
    st_ref[...] = jnp.maximum((st + 0.5).astype(jnp.int32) - 1, 0)


def _routing(logits, C, CH):
    S, E = logits.shape
    body = functools.partial(_k4_body, S=S, E=E, C=C, CH=CH)
    return pl.pallas_call(
        body,
        grid=(1,),
        in_specs=[pl.BlockSpec((S, E), lambda i: (0, 0))],
        out_specs=(
            pl.BlockSpec((C, E), lambda i: (0, 0)),
            pl.BlockSpec((2 * S, 1), lambda i: (0, 0)),
            pl.BlockSpec((2 * S, 1), lambda i: (0, 0)),
        ),
        out_shape=(
            jax.ShapeDtypeStruct((C, E), jnp.int32),
            jax.ShapeDtypeStruct((2 * S, 1), jnp.int32),
            jax.ShapeDtypeStruct((2 * S, 1), jnp.float32),
        ),
    )(logits)


def _k5_body(st_ref, hln_ref, w1_ref, w3_ref, w2_ref, eo_ref, *, BC, S, E):
    e = pl.program_id(0)
    cb = pl.program_id(1)
    blk = st_ref[pl.ds(cb * BC, BC), :]
    lane = jax.lax.broadcasted_iota(jnp.int32, (BC, E), 1)
    ids = jnp.sum(jnp.where(lane == e, blk, 0), axis=1, keepdims=True)
    iota_s = jax.lax.broadcasted_iota(jnp.int32, (BC, S), 1)
    disp = (ids == iota_s).astype(jnp.bfloat16)
    buf = jnp.dot(
        disp, hln_ref[...], preferred_element_type=jnp.float32
    ).astype(jnp.bfloat16)
    h1 = jnp.dot(buf, w1_ref[0], preferred_element_type=jnp.float32)
    h3 = jnp.dot(buf, w3_ref[0], preferred_element_type=jnp.float32)
    hh = (h1 * jax.lax.logistic(h1) * h3).astype(jnp.bfloat16)
    eo_ref[0] = jnp.dot(
        hh, w2_ref[0], preferred_element_type=jnp.float32
    ).astype(jnp.bfloat16)


def _expert_ffn(st, hln16, w1_16, w3_16, w2_16, BC):
    C, E = st.shape
    S, D = hln16.shape
    F = w1_16.shape[2]
    body = functools.partial(_k5_body, BC=BC, S=S, E=E)
    return pl.pallas_call(
        body,
        grid=(E, C // BC),
        in_specs=[
            pl.BlockSpec((C, E), lambda e, c: (0, 0)),
            pl.BlockSpec((S, D), lambda e, c: (0, 0)),
            pl.BlockSpec((1, D, F), lambda e, c: (e, 0, 0)),
            pl.BlockSpec((1, D, F), lambda e, c: (e, 0, 0)),
            pl.BlockSpec((1, F, D), lambda e, c: (e, 0, 0)),
        ],
        out_specs=pl.BlockSpec((1, BC, D), lambda e, c: (e, c, 0)),
        out_shape=jax.ShapeDtypeStruct((E, C, D), jnp.bfloat16),
    )(st, hln16, w1_16, w3_16, w2_16)


def _k6_body(cidx_ref, gate_ref, eo_ref, h_ref, o_ref, *, BT, S, NS):
    i = pl.program_id(0)
    c0 = cidx_ref[pl.ds(i * BT, BT), :]
    c1 = cidx_ref[pl.ds(S + i * BT, BT), :]
    g0 = gate_ref[pl.ds(i * BT, BT), :]
    g1 = gate_ref[pl.ds(S + i * BT, BT), :]
    iota_ns = jax.lax.broadcasted_iota(jnp.int32, (BT, NS), 1)
    G = (
        (c0 == iota_ns).astype(jnp.float32) * g0
        + (c1 == iota_ns).astype(jnp.float32) * g1
    ).astype(jnp.bfloat16)
    y = jnp.dot(G, eo_ref[...], preferred_element_type=jnp.float32)
    o_ref[...] = h_ref[...] + y


def _combine(cidx, gate, eo_flat, h, BT):
    NS, D = eo_flat.shape
    S = h.shape[0]
    body = functools.partial(_k6_body, BT=BT, S=S, NS=NS)
    return pl.pallas_call(
        body,
        grid=(S // BT,),
        in_specs=[
            pl.BlockSpec((2 * S, 1), lambda i: (0, 0)),
            pl.BlockSpec((2 * S, 1), lambda i: (0, 0)),
            pl.BlockSpec((NS, D), lambda i: (0, 0)),
            pl.BlockSpec((BT, D), lambda i: (i, 0)),
        ],
        out_specs=pl.BlockSpec((BT, D), lambda i: (i, 0)),
        out_shape=jax.ShapeDtypeStruct((S, D), jnp.float32),
    )(cidx, gate, eo_flat, h)


def kernel(x, ln1_w, Wq, Wk, Wv, Wo, ln2_w, Wr, W1, W3, W2):
    B, S, D = x.shape
    E = Wr.shape[1]
    C = int(math.ceil(B * S * TOP_K / E * CAP_FACTOR))
    HD = D // N_HEADS
    xf = x.reshape(S, D)

    bf = jnp.bfloat16
    q, k, v = _qkv(xf, ln1_w, Wq.astype(bf), Wk.astype(bf), Wv.astype(bf),
                   BS=256)
    qh = q.reshape(S, N_HEADS, HD).transpose(1, 0, 2)
    kh = k.reshape(S, N_HEADS, HD).transpose(1, 0, 2)
    vh = v.reshape(S, N_HEADS, HD).transpose(1, 0, 2)
    ao = _attention(qh, kh, vh, BQ=256)
    ao = ao.transpose(1, 0, 2).reshape(S, D)
    h, hln16, logits = _post_attn(ao, Wo.astype(bf), xf, ln2_w, Wr, BS=256)
    st, cidx, gate = _routing(logits, C=C, CH=min(512, 2 * S))
    eo = _expert_ffn(st, hln16, W1.astype(bf), W3.astype(bf), W2.astype(bf),
                     BC=C // 5)
    out = _combine(cidx, gate, eo.reshape(E * C, D), h, BT=256)
    return out.reshape(B, S, D)


# speedup vs baseline: 1.1553x; 1.0739x over previous
import functools
import math

import jax
import jax.numpy as jnp
from jax import lax
from jax.experimental import pallas as pl
from jax.experimental.pallas import tpu as pltpu
from jax.experimental.pallas import tpu_sc as plsc

EPS = 1e-5
N_HEADS = 16
TOP_K = 2
CAP_FACTOR = 1.25
NEG_INF = -1e30


def _k1_body(x_ref, w_ref, wq_ref, wk_ref, wv_ref, q_ref, k_ref, v_ref):
    x = x_ref[...]
    var = jnp.mean(x * x, axis=-1, keepdims=True)
    xln = (x * jax.lax.rsqrt(var + EPS) * w_ref[...]).astype(jnp.bfloat16)
    for wr, outr in ((wq_ref, q_ref), (wk_ref, k_ref), (wv_ref, v_ref)):
        outr[...] = jnp.dot(
            xln, wr[...].astype(jnp.bfloat16),
            preferred_element_type=jnp.float32,
        ).astype(jnp.bfloat16)


def _qkv(x, ln1_w, wq16, wk16, wv16, BS):
    S, D = x.shape
    row = pl.BlockSpec((BS, D), lambda i: (i, 0))
    full = pl.BlockSpec((D, D), lambda i: (0, 0))
    wspec = pl.BlockSpec((1, D), lambda i: (0, 0))
    out = jax.ShapeDtypeStruct((S, D), jnp.bfloat16)
    return pl.pallas_call(
        _k1_body,
        grid=(S // BS,),
        in_specs=[row, wspec, full, full, full],
        out_specs=(row, row, row),
        out_shape=(out, out, out),
    )(x, ln1_w.reshape(1, D), wq16, wk16, wv16)


def _k2_body(q_ref, k_ref, v_ref, o_ref, *, BQ, BK, HD, scale):
    qb = pl.program_id(1)
    q2 = q_ref[...]
    row = qb * BQ + jax.lax.broadcasted_iota(jnp.int32, (BQ, BK), 0)

    def step(j, carry):
        ma, la, acca, mb, lb, accb = carry
        kc = k_ref[pl.ds(j * BK, BK), :]
        vc = v_ref[pl.ds(j * BK, BK), :]
        col = j * BK + jax.lax.broadcasted_iota(jnp.int32, (BQ, BK), 1)
        mask = col <= row

        def upd(qh_, kh_, vh_, m, l, acc):
            s = jax.lax.dot_general(
                qh_, kh_, (((1,), (1,)), ((), ())),
                preferred_element_type=jnp.float32,
            ) * scale
            s = jnp.where(mask, s, NEG_INF)
            mn = jnp.maximum(m, jnp.max(s, axis=-1, keepdims=True))
            p = jnp.exp(s - mn)
            corr = jnp.exp(m - mn)
            l = l * corr + jnp.sum(p, axis=-1, keepdims=True)
            acc = acc * corr + jnp.dot(
                p.astype(jnp.bfloat16), vh_,
                preferred_element_type=jnp.float32,
            )
            return mn, l, acc

        ma, la, acca = upd(q2[:, :HD], kc[:, :HD], vc[:, :HD], ma, la, acca)
        mb, lb, accb = upd(q2[:, HD:], kc[:, HD:], vc[:, HD:], mb, lb, accb)
        return ma, la, acca, mb, lb, accb

    m0 = jnp.full((BQ, 1), NEG_INF, jnp.float32)
    l0 = jnp.zeros((BQ, 1), jnp.float32)
    a0 = jnp.zeros((BQ, HD), jnp.float32)
    ma, la, acca, mb, lb, accb = jax.lax.fori_loop(
        0, qb + 1, step, (m0, l0, a0, m0, l0, a0)
    )
    o_ref[...] = jnp.concatenate(
        [acca / la, accb / lb], axis=1
    ).astype(jnp.bfloat16)


def _attention(q, k, v, n_heads, BQ):
    S, D = q.shape
    HD = D // n_heads
    HP = n_heads // 2
    body = functools.partial(
        _k2_body, BQ=BQ, BK=BQ, HD=HD, scale=1.0 / math.sqrt(HD)
    )
    return pl.pallas_call(
        body,
        grid=(HP, S // BQ),
        in_specs=[
            pl.BlockSpec((BQ, 2 * HD), lambda hp, i: (i, hp)),
            pl.BlockSpec((S, 2 * HD), lambda hp, i: (0, hp)),
            pl.BlockSpec((S, 2 * HD), lambda hp, i: (0, hp)),
        ],
        out_specs=pl.BlockSpec((BQ, 2 * HD), lambda hp, i: (i, hp)),
        out_shape=jax.ShapeDtypeStruct((S, D), jnp.bfloat16),
    )(q, k, v)


def _k3_body(ao_ref, wo_ref, x_ref, w2_ref, wr_ref, h_ref, hln_ref, lg_ref):
    att = jnp.dot(
        ao_ref[...], wo_ref[...].astype(jnp.bfloat16),
        preferred_element_type=jnp.float32,
    )
    h = att + x_ref[...]
    h_ref[...] = h
    var = jnp.mean(h * h, axis=-1, keepdims=True)
    hln = h * jax.lax.rsqrt(var + EPS) * w2_ref[...]
    hln_ref[...] = hln
    lg_ref[...] = jnp.dot(
        hln,
        wr_ref[...],
        preferred_element_type=jnp.float32,
        precision=jax.lax.Precision.HIGHEST,
    )


def _post_attn(ao, wo16, x, ln2_w, wr, BS):
    S, D = x.shape
    E = wr.shape[1]
    row = pl.BlockSpec((BS, D), lambda i: (i, 0))
    return pl.pallas_call(
        _k3_body,
        grid=(S // BS,),
        in_specs=[
            row,
            pl.BlockSpec((D, D), lambda i: (0, 0)),
            row,
            pl.BlockSpec((1, D), lambda i: (0, 0)),
            pl.BlockSpec((D, E), lambda i: (0, 0)),
        ],
        out_specs=(row, row, pl.BlockSpec((BS, E), lambda i: (i, 0))),
        out_shape=(
            jax.ShapeDtypeStruct((S, D), jnp.float32),
            jax.ShapeDtypeStruct((S, D), jnp.float32),
            jax.ShapeDtypeStruct((S, E), jnp.float32),
        ),
    )(ao, wo16, x, ln2_w.reshape(1, D), wr)


def _k4_body(lg_ref, st_ref, cidx_ref, gate_ref, *, S, E, C, CH):
    lg = lg_ref[...]
    m = jnp.max(lg, axis=-1, keepdims=True)
    ex = jnp.exp(lg - m)
    probs = ex / jnp.sum(ex, axis=-1, keepdims=True)

    iota_e = jax.lax.broadcasted_iota(jnp.int32, (S, E), 1)
    m1 = jnp.max(probs, axis=-1, keepdims=True)
    i1 = jnp.min(jnp.where(probs == m1, iota_e, E), axis=-1, keepdims=True)
    probs2 = jnp.where(iota_e == i1, -1.0, probs)
    m2 = jnp.max(probs2, axis=-1, keepdims=True)
    i2 = jnp.min(jnp.where(probs2 == m2, iota_e, E), axis=-1, keepdims=True)
    den = m1 + m2 + 1e-9
    g1 = m1 / den
    g2 = m2 / den

    oh0 = (iota_e == i1).astype(jnp.float32)
    oh1 = (iota_e == i2).astype(jnp.float32)
    A = jnp.concatenate([oh0, oh1], axis=0)

    tri = (
        jax.lax.broadcasted_iota(jnp.int32, (CH, CH), 0)
        > jax.lax.broadcasted_iota(jnp.int32, (CH, CH), 1)
    ).astype(jnp.float32)
    carry = jnp.zeros((1, E), jnp.float32)
    prefs = []
    for j in range(2 * S // CH):
        blk = jax.lax.slice(A, (j * CH, 0), ((j + 1) * CH, E))
        prefs.append(
            jnp.dot(
                tri,
                blk,
                preferred_element_type=jnp.float32,
                precision=jax.lax.Precision.HIGHEST,
            )
            + carry
        )
        carry = carry + jnp.sum(blk, axis=0, keepdims=True)
    pos = jnp.concatenate(prefs, axis=0)
    pos_sel = jnp.floor(
        jnp.sum(pos * A, axis=-1, keepdims=True) + 0.5
    )
    valid = (pos_sel < C).astype(jnp.float32)
    pos_c = jnp.clip(pos_sel, 0.0, C - 1.0)

    gate_all = jnp.concatenate([g1, g2], axis=0) * valid
    e_all = jnp.concatenate([i1, i2], axis=0).astype(jnp.float32)
    cidx_ref[...] = (e_all * C + pos_c).astype(jnp.int32)
    gate_ref[...] = gate_all

    iota_c = jax.lax.broadcasted_iota(jnp.int32, (2 * S, C), 1)
    P = (pos_c.astype(jnp.int32) == iota_c).astype(jnp.float32) * valid
    t_all = jax.lax.broadcasted_iota(jnp.int32, (2 * S, 1), 0).astype(
        jnp.float32
    )
    t_all = jnp.where(t_all < S, t_all, t_all - S)
    lhs = A * ((t_all + 1.0) * valid)
    st = jax.lax.dot_general(
        P,
        lhs,
        (((0,), (0,)), ((), ())),
        preferred_element_type=jnp.float32,
        precision=jax.lax.Precision.HIGHEST,
    )
    st_ref[...] = jnp.maximum((st + 0.5).astype(jnp.int32) - 1, 0)


def _routing(logits, C, CH):
    S, E = logits.shape
    body = functools.partial(_k4_body, S=S, E=E, C=C, CH=CH)
    return pl.pallas_call(
        body,
        grid=(1,),
        in_specs=[pl.BlockSpec((S, E), lambda i: (0, 0))],
        out_specs=(
            pl.BlockSpec((C, E), lambda i: (0, 0)),
            pl.BlockSpec((2 * S, 1), lambda i: (0, 0)),
            pl.BlockSpec((2 * S, 1), lambda i: (0, 0)),
        ),
        out_shape=(
            jax.ShapeDtypeStruct((C, E), jnp.int32),
            jax.ShapeDtypeStruct((2 * S, 1), jnp.int32),
            jax.ShapeDtypeStruct((2 * S, 1), jnp.float32),
        ),
    )(logits)


def _sc_dispatch(hln, st3d):
    S, D = hln.shape
    NW, NCH, CHW = st3d.shape
    mesh = plsc.VectorSubcoreMesh(
        core_axis_name="c", subcore_axis_name="s",
        num_cores=2, num_subcores=16,
    )

    @functools.partial(
        pl.kernel,
        mesh=mesh,
        out_type=jax.ShapeDtypeStruct((NW * NCH * CHW, D), jnp.float32),
        scratch_types=[
            pltpu.VMEM((NCH, CHW), jnp.int32),
            pltpu.VMEM((CHW, D), jnp.float32),
            pltpu.SemaphoreType.DMA,
        ],
    )
    def k(hln_hbm, st_hbm, buf_hbm, idx_v, rows_v, sem):
        cid = lax.axis_index("c")
        sid = lax.axis_index("s")
        wid = sid * 2 + cid
        pltpu.sync_copy(st_hbm.at[wid], idx_v)
        for j in range(NCH):
            pltpu.async_copy(hln_hbm.at[idx_v.at[j]], rows_v, sem).wait()
            pltpu.sync_copy(
                rows_v, buf_hbm.at[pl.ds(wid * NCH * CHW + j * CHW, CHW)]
            )

    return k(hln, st3d)


def _k5_body(buf_ref, w1_ref, w3_ref, w2_ref, eo_ref):
    buf = buf_ref[...].astype(jnp.bfloat16)
    h1 = jnp.dot(buf, w1_ref[0].astype(jnp.bfloat16),
                 preferred_element_type=jnp.float32)
    h3 = jnp.dot(buf, w3_ref[0].astype(jnp.bfloat16),
                 preferred_element_type=jnp.float32)
    hh = (h1 * jax.lax.logistic(h1) * h3).astype(jnp.bfloat16)
    eo_ref[...] = jnp.dot(hh, w2_ref[0].astype(jnp.bfloat16),
                          preferred_element_type=jnp.float32)


def _expert_ffn(buf, E, C, w1_16, w3_16, w2_16, BC):
    NSLOT, D = buf.shape
    F = w1_16.shape[2]
    nb = C // BC
    return pl.pallas_call(
        _k5_body,
        grid=(E, nb),
        in_specs=[
            pl.BlockSpec((BC, D), lambda e, c: (e * nb + c, 0)),
            pl.BlockSpec((1, D, F), lambda e, c: (e, 0, 0)),
            pl.BlockSpec((1, D, F), lambda e, c: (e, 0, 0)),
            pl.BlockSpec((1, F, D), lambda e, c: (e, 0, 0)),
        ],
        out_specs=pl.BlockSpec((BC, D), lambda e, c: (e * nb + c, 0)),
        out_shape=jax.ShapeDtypeStruct((NSLOT, D), jnp.float32),
    )(buf, w1_16, w3_16, w2_16)


def _sc_combine_gather(eo, cidx3d):
    NSLOT, D = eo.shape
    NW, KK, TPW = cidx3d.shape
    S = NW * TPW
    mesh = plsc.VectorSubcoreMesh(
        core_axis_name="c", subcore_axis_name="s",
        num_cores=2, num_subcores=16,
    )

    @functools.partial(
        pl.kernel,
        mesh=mesh,
        out_type=jax.ShapeDtypeStruct((KK, S, D), jnp.float32),
        scratch_types=[
            pltpu.VMEM((KK, TPW), jnp.int32),
            pltpu.VMEM((TPW, D), jnp.float32),
            pltpu.SemaphoreType.DMA,
        ],
    )
    def k(eo_hbm, cidx_hbm, r_hbm, idx_v, rows_v, sem):
        cid = lax.axis_index("c")
        sid = lax.axis_index("s")
        wid = sid * 2 + cid
        pltpu.sync_copy(cidx_hbm.at[wid], idx_v)
        for j in range(KK):
            pltpu.async_copy(eo_hbm.at[idx_v.at[j]], rows_v, sem).wait()
            pltpu.sync_copy(rows_v, r_hbm.at[j, pl.ds(wid * TPW, TPW)])

    return k(eo, cidx3d)


def _k7_body(r_ref, gate_ref, h_ref, o_ref, *, BT, S):
    i = pl.program_id(0)
    g0 = gate_ref[pl.ds(i * BT, BT), :]
    g1 = gate_ref[pl.ds(S + i * BT, BT), :]
    o_ref[...] = h_ref[...] + g0 * r_ref[0] + g1 * r_ref[1]


def _combine_add(r, gate, h, BT):
    KK, S, D = r.shape
    body = functools.partial(_k7_body, BT=BT, S=S)
    return pl.pallas_call(
        body,
        grid=(S // BT,),
        in_specs=[
            pl.BlockSpec((KK, BT, D), lambda i: (0, i, 0)),
            pl.BlockSpec((2 * S, 1), lambda i: (0, 0)),
            pl.BlockSpec((BT, D), lambda i: (i, 0)),
        ],
        out_specs=pl.BlockSpec((BT, D), lambda i: (i, 0)),
        out_shape=jax.ShapeDtypeStruct((S, D), jnp.float32),
    )(r, gate, h)


def kernel(x, ln1_w, Wq, Wk, Wv, Wo, ln2_w, Wr, W1, W3, W2):
    B, S, D = x.shape
    E = Wr.shape[1]
    C = int(math.ceil(B * S * TOP_K / E * CAP_FACTOR))
    HD = D // N_HEADS
    xf = x.reshape(S, D)

    q, k, v = _qkv(xf, ln1_w, Wq, Wk, Wv, BS=256)
    ao = _attention(q, k, v, N_HEADS, BQ=256)
    h, hln, logits = _post_attn(ao, Wo, xf, ln2_w, Wr, BS=256)
    st, cidx, gate = _routing(logits, C=C, CH=min(512, 2 * S))
    st_flat = st.T.reshape(-1)
    NSLOT = E * C
    rpw = NSLOT // 32
    nch = -(-rpw // 80)
    buf = _sc_dispatch(hln, st_flat.reshape(32, nch, rpw // nch))
    eo = _expert_ffn(buf, E, C, W1, W3, W2, BC=C // 5)
    cidx3d = cidx.reshape(2, 32, S // 32).transpose(1, 0, 2)
    r = _sc_combine_gather(eo, cidx3d)
    out = _combine_add(r, gate, h, BT=256)
    return out.reshape(B, S, D)


# speedup vs baseline: 1.4880x; 1.2879x over previous
import functools
import math

import jax
import jax.numpy as jnp
from jax import lax
from jax.experimental import pallas as pl
from jax.experimental.pallas import tpu as pltpu
from jax.experimental.pallas import tpu_sc as plsc

EPS = 1e-5
N_HEADS = 16
TOP_K = 2
CAP_FACTOR = 1.25
NEG_INF = -1e30


def _k1_body(x_ref, w_ref, wq_ref, wk_ref, wv_ref, q_ref, k_ref, v_ref):
    x = x_ref[...]
    var = jnp.mean(x * x, axis=-1, keepdims=True)
    xln = (x * jax.lax.rsqrt(var + EPS) * w_ref[...]).astype(jnp.bfloat16)
    for wr, outr in ((wq_ref, q_ref), (wk_ref, k_ref), (wv_ref, v_ref)):
        outr[...] = jnp.dot(
            xln, wr[...].astype(jnp.bfloat16),
            preferred_element_type=jnp.float32,
        ).astype(jnp.bfloat16)


def _qkv(x, ln1_w, wq16, wk16, wv16, BS):
    S, D = x.shape
    row = pl.BlockSpec((BS, D), lambda i: (i, 0))
    full = pl.BlockSpec((D, D), lambda i: (0, 0))
    wspec = pl.BlockSpec((1, D), lambda i: (0, 0))
    out = jax.ShapeDtypeStruct((S, D), jnp.bfloat16)
    return pl.pallas_call(
        _k1_body,
        grid=(S // BS,),
        in_specs=[row, wspec, full, full, full],
        out_specs=(row, row, row),
        out_shape=(out, out, out),
    )(x, ln1_w.reshape(1, D), wq16, wk16, wv16)


def _k2_body(q_ref, k_ref, v_ref, o_ref, *, BQ, BK, HD, scale):
    qb = pl.program_id(1)
    q2 = q_ref[...]
    row = qb * BQ + jax.lax.broadcasted_iota(jnp.int32, (BQ, BK), 0)

    def step(j, carry):
        ma, la, acca, mb, lb, accb = carry
        kc = k_ref[pl.ds(j * BK, BK), :]
        vc = v_ref[pl.ds(j * BK, BK), :]
        col = j * BK + jax.lax.broadcasted_iota(jnp.int32, (BQ, BK), 1)
        mask = col <= row

        def upd(qh_, kh_, vh_, m, l, acc):
            s = jax.lax.dot_general(
                qh_, kh_, (((1,), (1,)), ((), ())),
                preferred_element_type=jnp.float32,
            ) * scale
            s = jnp.where(mask, s, NEG_INF)
            mn = jnp.maximum(m, jnp.max(s, axis=-1, keepdims=True))
            p = jnp.exp(s - mn)
            corr = jnp.exp(m - mn)
            l = l * corr + jnp.sum(p, axis=-1, keepdims=True)
            acc = acc * corr + jnp.dot(
                p.astype(jnp.bfloat16), vh_,
                preferred_element_type=jnp.float32,
            )
            return mn, l, acc

        ma, la, acca = upd(q2[:, :HD], kc[:, :HD], vc[:, :HD], ma, la, acca)
        mb, lb, accb = upd(q2[:, HD:], kc[:, HD:], vc[:, HD:], mb, lb, accb)
        return ma, la, acca, mb, lb, accb

    m0 = jnp.full((BQ, 1), NEG_INF, jnp.float32)
    l0 = jnp.zeros((BQ, 1), jnp.float32)
    a0 = jnp.zeros((BQ, HD), jnp.float32)
    ma, la, acca, mb, lb, accb = jax.lax.fori_loop(
        0, qb + 1, step, (m0, l0, a0, m0, l0, a0)
    )
    o_ref[...] = jnp.concatenate(
        [acca / la, accb / lb], axis=1
    ).astype(jnp.bfloat16)


def _attention(q, k, v, n_heads, BQ):
    S, D = q.shape
    HD = D // n_heads
    HP = n_heads // 2
    body = functools.partial(
        _k2_body, BQ=BQ, BK=BQ, HD=HD, scale=1.0 / math.sqrt(HD)
    )
    return pl.pallas_call(
        body,
        grid=(HP, S // BQ),
        in_specs=[
            pl.BlockSpec((BQ, 2 * HD), lambda hp, i: (i, hp)),
            pl.BlockSpec((S, 2 * HD), lambda hp, i: (0, hp)),
            pl.BlockSpec((S, 2 * HD), lambda hp, i: (0, hp)),
        ],
        out_specs=pl.BlockSpec((BQ, 2 * HD), lambda hp, i: (i, hp)),
        out_shape=jax.ShapeDtypeStruct((S, D), jnp.bfloat16),
    )(q, k, v)


def _k3_body(ao_ref, wo_ref, x_ref, w2_ref, wr_ref, h_ref, hln_ref, lg_ref):
    att = jnp.dot(
        ao_ref[...], wo_ref[...].astype(jnp.bfloat16),
        preferred_element_type=jnp.float32,
    )
    h = att + x_ref[...]
    h_ref[...] = h
    var = jnp.mean(h * h, axis=-1, keepdims=True)
    hln = h * jax.lax.rsqrt(var + EPS) * w2_ref[...]
    hln_ref[...] = hln
    lg_ref[...] = jnp.dot(
        hln,
        wr_ref[...],
        preferred_element_type=jnp.float32,
        precision=jax.lax.Precision.HIGHEST,
    )


def _post_attn(ao, wo16, x, ln2_w, wr, BS):
    S, D = x.shape
    E = wr.shape[1]
    row = pl.BlockSpec((BS, D), lambda i: (i, 0))
    return pl.pallas_call(
        _k3_body,
        grid=(S // BS,),
        in_specs=[
            row,
            pl.BlockSpec((D, D), lambda i: (0, 0)),
            row,
            pl.BlockSpec((1, D), lambda i: (0, 0)),
            pl.BlockSpec((D, E), lambda i: (0, 0)),
        ],
        out_specs=(row, row, pl.BlockSpec((BS, E), lambda i: (i, 0))),
        out_shape=(
            jax.ShapeDtypeStruct((S, D), jnp.float32),
            jax.ShapeDtypeStruct((S, D), jnp.float32),
            jax.ShapeDtypeStruct((S, E), jnp.float32),
        ),
    )(ao, wo16, x, ln2_w.reshape(1, D), wr)


def _k4_body(lg_ref, st_ref, cidx_ref, gate_ref, *, S, E, C, CH):
    lg = lg_ref[...]
    m = jnp.max(lg, axis=-1, keepdims=True)
    ex = jnp.exp(lg - m)
    probs = ex / jnp.sum(ex, axis=-1, keepdims=True)

    iota_e = jax.lax.broadcasted_iota(jnp.int32, (S, E), 1)
    m1 = jnp.max(probs, axis=-1, keepdims=True)
    i1 = jnp.min(jnp.where(probs == m1, iota_e, E), axis=-1, keepdims=True)
    probs2 = jnp.where(iota_e == i1, -1.0, probs)
    m2 = jnp.max(probs2, axis=-1, keepdims=True)
    i2 = jnp.min(jnp.where(probs2 == m2, iota_e, E), axis=-1, keepdims=True)
    den = m1 + m2 + 1e-9
    g1 = m1 / den
    g2 = m2 / den

    oh0 = (iota_e == i1).astype(jnp.float32)
    oh1 = (iota_e == i2).astype(jnp.float32)
    A = jnp.concatenate([oh0, oh1], axis=0)

    tri = (
        jax.lax.broadcasted_iota(jnp.int32, (CH, CH), 0)
        > jax.lax.broadcasted_iota(jnp.int32, (CH, CH), 1)
    ).astype(jnp.float32)
    carry = jnp.zeros((1, E), jnp.float32)
    prefs = []
    for j in range(2 * S // CH):
        blk = jax.lax.slice(A, (j * CH, 0), ((j + 1) * CH, E))
        prefs.append(
            jnp.dot(
                tri,
                blk,
                preferred_element_type=jnp.float32,
                precision=jax.lax.Precision.HIGHEST,
            )
            + carry
        )
        carry = carry + jnp.sum(blk, axis=0, keepdims=True)
    pos = jnp.concatenate(prefs, axis=0)
    pos_sel = jnp.floor(
        jnp.sum(pos * A, axis=-1, keepdims=True) + 0.5
    )
    valid = (pos_sel < C).astype(jnp.float32)
    pos_c = jnp.clip(pos_sel, 0.0, C - 1.0)

    gate_all = jnp.concatenate([g1, g2], axis=0) * valid
    e_all = jnp.concatenate([i1, i2], axis=0).astype(jnp.float32)
    cidx_ref[...] = (e_all * C + pos_c).astype(jnp.int32)
    gate_ref[...] = gate_all

    iota_c = jax.lax.broadcasted_iota(jnp.int32, (2 * S, C), 1)
    P = (pos_c.astype(jnp.int32) == iota_c).astype(jnp.float32) * valid
    t_all = jax.lax.broadcasted_iota(jnp.int32, (2 * S, 1), 0).astype(
        jnp.float32
    )
    t_all = jnp.where(t_all < S, t_all, t_all - S)
    lhs = A * ((t_all + 1.0) * valid)
    st = jax.lax.dot_general(
        lhs,
        P,
        (((0,), (0,)), ((), ())),
        preferred_element_type=jnp.float32,
        precision=jax.lax.Precision.HIGHEST,
    )
    st_ref[...] = jnp.maximum((st + 0.5).astype(jnp.int32) - 1, 0)


def _routing(logits, C, CH):
    S, E = logits.shape
    body = functools.partial(_k4_body, S=S, E=E, C=C, CH=CH)
    return pl.pallas_call(
        body,
        grid=(1,),
        in_specs=[pl.BlockSpec((S, E), lambda i: (0, 0))],
        out_specs=(
            pl.BlockSpec((E, C), lambda i: (0, 0)),
            pl.BlockSpec((2 * S, 1), lambda i: (0, 0)),
            pl.BlockSpec((2 * S, 1), lambda i: (0, 0)),
        ),
        out_shape=(
            jax.ShapeDtypeStruct((E, C), jnp.int32),
            jax.ShapeDtypeStruct((2 * S, 1), jnp.int32),
            jax.ShapeDtypeStruct((2 * S, 1), jnp.float32),
        ),
    )(logits)


def _sc_dispatch(hln, st3d):
    S, D = hln.shape
    NW, NCH, CHW = st3d.shape
    mesh = plsc.VectorSubcoreMesh(
        core_axis_name="c", subcore_axis_name="s",
        num_cores=2, num_subcores=16,
    )

    @functools.partial(
        pl.kernel,
        mesh=mesh,
        out_type=jax.ShapeDtypeStruct((NW * NCH * CHW, D), jnp.float32),
        scratch_types=[
            pltpu.VMEM((NCH, CHW), jnp.int32),
            pltpu.VMEM((CHW, D), jnp.float32),
            pltpu.SemaphoreType.DMA,
        ],
    )
    def k(hln_hbm, st_hbm, buf_hbm, idx_v, rows_v, sem):
        cid = lax.axis_index("c")
        sid = lax.axis_index("s")
        wid = sid * 2 + cid
        pltpu.sync_copy(st_hbm.at[wid], idx_v)
        for j in range(NCH):
            pltpu.async_copy(hln_hbm.at[idx_v.at[j]], rows_v, sem).wait()
            pltpu.sync_copy(
                rows_v, buf_hbm.at[pl.ds(wid * NCH * CHW + j * CHW, CHW)]
            )

    return k(hln, st3d)


def _k5_body(buf_ref, w1_ref, w3_ref, w2_ref, eo_ref):
    buf = buf_ref[...].astype(jnp.bfloat16)
    h1 = jnp.dot(buf, w1_ref[0].astype(jnp.bfloat16),
                 preferred_element_type=jnp.float32)
    h3 = jnp.dot(buf, w3_ref[0].astype(jnp.bfloat16),
                 preferred_element_type=jnp.float32)
    hh = (h1 * jax.lax.logistic(h1) * h3).astype(jnp.bfloat16)
    eo_ref[...] = jnp.dot(hh, w2_ref[0].astype(jnp.bfloat16),
                          preferred_element_type=jnp.float32)


def _expert_ffn(buf, E, C, w1_16, w3_16, w2_16, BC):
    NSLOT, D = buf.shape
    F = w1_16.shape[2]
    nb = C // BC
    return pl.pallas_call(
        _k5_body,
        grid=(E, nb),
        in_specs=[
            pl.BlockSpec((BC, D), lambda e, c: (e * nb + c, 0)),
            pl.BlockSpec((1, D, F), lambda e, c: (e, 0, 0)),
            pl.BlockSpec((1, D, F), lambda e, c: (e, 0, 0)),
            pl.BlockSpec((1, F, D), lambda e, c: (e, 0, 0)),
        ],
        out_specs=pl.BlockSpec((BC, D), lambda e, c: (e * nb + c, 0)),
        out_shape=jax.ShapeDtypeStruct((NSLOT, D), jnp.float32),
    )(buf, w1_16, w3_16, w2_16)


def _sc_combine_gather(eo, cidx3d):
    NSLOT, D = eo.shape
    KK, NW, TPW = cidx3d.shape
    S = NW * TPW
    mesh = plsc.VectorSubcoreMesh(
        core_axis_name="c", subcore_axis_name="s",
        num_cores=2, num_subcores=16,
    )

    @functools.partial(
        pl.kernel,
        mesh=mesh,
        out_type=jax.ShapeDtypeStruct((KK, S, D), jnp.float32),
        scratch_types=[
            pltpu.VMEM((KK, TPW), jnp.int32),
            pltpu.VMEM((TPW, D), jnp.float32),
            pltpu.SemaphoreType.DMA,
        ],
    )
    def k(eo_hbm, cidx_hbm, r_hbm, idx_v, rows_v, sem):
        cid = lax.axis_index("c")
        sid = lax.axis_index("s")
        wid = sid * 2 + cid
        for j in range(KK):
            pltpu.sync_copy(cidx_hbm.at[j, wid], idx_v.at[j])
        for j in range(KK):
            pltpu.async_copy(eo_hbm.at[idx_v.at[j]], rows_v, sem).wait()
            pltpu.sync_copy(rows_v, r_hbm.at[j, pl.ds(wid * TPW, TPW)])

    return k(eo, cidx3d)


def _k7_body(r_ref, gate_ref, h_ref, o_ref, *, BT, S):
    i = pl.program_id(0)
    g0 = gate_ref[pl.ds(i * BT, BT), :]
    g1 = gate_ref[pl.ds(S + i * BT, BT), :]
    o_ref[...] = h_ref[...] + g0 * r_ref[0] + g1 * r_ref[1]


def _combine_add(r, gate, h, BT):
    KK, S, D = r.shape
    body = functools.partial(_k7_body, BT=BT, S=S)
    return pl.pallas_call(
        body,
        grid=(S // BT,),
        in_specs=[
            pl.BlockSpec((KK, BT, D), lambda i: (0, i, 0)),
            pl.BlockSpec((2 * S, 1), lambda i: (0, 0)),
            pl.BlockSpec((BT, D), lambda i: (i, 0)),
        ],
        out_specs=pl.BlockSpec((BT, D), lambda i: (i, 0)),
        out_shape=jax.ShapeDtypeStruct((S, D), jnp.float32),
    )(r, gate, h)


def kernel(x, ln1_w, Wq, Wk, Wv, Wo, ln2_w, Wr, W1, W3, W2):
    B, S, D = x.shape
    E = Wr.shape[1]
    C = int(math.ceil(B * S * TOP_K / E * CAP_FACTOR))
    HD = D // N_HEADS
    xf = x.reshape(S, D)

    q, k, v = _qkv(xf, ln1_w, Wq, Wk, Wv, BS=256)
    ao = _attention(q, k, v, N_HEADS, BQ=min(512, S))
    h, hln, logits = _post_attn(ao, Wo, xf, ln2_w, Wr, BS=256)
    st, cidx, gate = _routing(logits, C=C, CH=min(512, 2 * S))
    st_flat = st.reshape(-1)
    NSLOT = E * C
    rpw = NSLOT // 32
    nch = -(-rpw // 80)
    buf = _sc_dispatch(hln, st_flat.reshape(32, nch, rpw // nch))
    eo = _expert_ffn(buf, E, C, W1, W3, W2, BC=C // 5)
    cidx3d = cidx.reshape(2, 32, S // 32)
    r = _sc_combine_gather(eo, cidx3d)
    out = _combine_add(r, gate, h, BT=256)
    return out.reshape(B, S, D)


# speedup vs baseline: 1.6688x; 1.1216x over previous
import functools
import math

import jax
import jax.numpy as jnp
from jax import lax
from jax.experimental import pallas as pl
from jax.experimental.pallas import tpu as pltpu
from jax.experimental.pallas import tpu_sc as plsc

EPS = 1e-5
N_HEADS = 16
TOP_K = 2
CAP_FACTOR = 1.25
NEG_INF = -1e30


def _k1_body(x_ref, w_ref, wq_ref, wk_ref, wv_ref, q_ref, k_ref, v_ref):
    x = x_ref[...]
    var = jnp.mean(x * x, axis=-1, keepdims=True)
    xln = (x * jax.lax.rsqrt(var + EPS) * w_ref[...]).astype(jnp.bfloat16)
    for wr, outr in ((wq_ref, q_ref), (wk_ref, k_ref), (wv_ref, v_ref)):
        outr[...] = jnp.dot(
            xln, wr[...].astype(jnp.bfloat16),
            preferred_element_type=jnp.float32,
        ).astype(jnp.bfloat16)


def _qkv(x, ln1_w, wq16, wk16, wv16, BS):
    S, D = x.shape
    row = pl.BlockSpec((BS, D), lambda i: (i, 0))
    full = pl.BlockSpec((D, D), lambda i: (0, 0))
    wspec = pl.BlockSpec((1, D), lambda i: (0, 0))
    out = jax.ShapeDtypeStruct((S, D), jnp.bfloat16)
    return pl.pallas_call(
        _k1_body,
        grid=(S // BS,),
        in_specs=[row, wspec, full, full, full],
        out_specs=(row, row, row),
        out_shape=(out, out, out),
    )(x, ln1_w.reshape(1, D), wq16, wk16, wv16)


def _k2_body(q_ref, k_ref, v_ref, o_ref, *, BQ, BK, HD, scale):
    qb = pl.program_id(1)
    q2 = q_ref[...]
    row = qb * BQ + jax.lax.broadcasted_iota(jnp.int32, (BQ, BK), 0)

    def step(j, carry):
        ma, la, acca, mb, lb, accb = carry
        kc = k_ref[pl.ds(j * BK, BK), :]
        vc = v_ref[pl.ds(j * BK, BK), :]
        col = j * BK + jax.lax.broadcasted_iota(jnp.int32, (BQ, BK), 1)
        mask = col <= row

        def upd(qh_, kh_, vh_, m, l, acc):
            s = jax.lax.dot_general(
                qh_, kh_, (((1,), (1,)), ((), ())),
                preferred_element_type=jnp.float32,
            ) * scale
            s = jnp.where(mask, s, NEG_INF)
            mn = jnp.maximum(m, jnp.max(s, axis=-1, keepdims=True))
            p = jnp.exp(s - mn)
            corr = jnp.exp(m - mn)
            l = l * corr + jnp.sum(p, axis=-1, keepdims=True)
            acc = acc * corr + jnp.dot(
                p.astype(jnp.bfloat16), vh_,
                preferred_element_type=jnp.float32,
            )
            return mn, l, acc

        ma, la, acca = upd(q2[:, :HD], kc[:, :HD], vc[:, :HD], ma, la, acca)
        mb, lb, accb = upd(q2[:, HD:], kc[:, HD:], vc[:, HD:], mb, lb, accb)
        return ma, la, acca, mb, lb, accb

    m0 = jnp.full((BQ, 1), NEG_INF, jnp.float32)
    l0 = jnp.zeros((BQ, 1), jnp.float32)
    a0 = jnp.zeros((BQ, HD), jnp.float32)
    ma, la, acca, mb, lb, accb = jax.lax.fori_loop(
        0, qb + 1, step, (m0, l0, a0, m0, l0, a0)
    )
    o_ref[...] = jnp.concatenate(
        [acca / la, accb / lb], axis=1
    ).astype(jnp.bfloat16)


def _attention(q, k, v, n_heads, BQ):
    S, D = q.shape
    HD = D // n_heads
    HP = n_heads // 2
    body = functools.partial(
        _k2_body, BQ=BQ, BK=BQ, HD=HD, scale=1.0 / math.sqrt(HD)
    )
    return pl.pallas_call(
        body,
        grid=(HP, S // BQ),
        in_specs=[
            pl.BlockSpec((BQ, 2 * HD), lambda hp, i: (i, hp)),
            pl.BlockSpec((S, 2 * HD), lambda hp, i: (0, hp)),
            pl.BlockSpec((S, 2 * HD), lambda hp, i: (0, hp)),
        ],
        out_specs=pl.BlockSpec((BQ, 2 * HD), lambda hp, i: (i, hp)),
        out_shape=jax.ShapeDtypeStruct((S, D), jnp.bfloat16),
    )(q, k, v)


def _k3_body(ao_ref, wo_ref, x_ref, w2_ref, wr_ref, h_ref, hln_ref, lg_ref):
    att = jnp.dot(
        ao_ref[...], wo_ref[...].astype(jnp.bfloat16),
        preferred_element_type=jnp.float32,
    )
    h = att + x_ref[...]
    h_ref[...] = h
    var = jnp.mean(h * h, axis=-1, keepdims=True)
    hln = h * jax.lax.rsqrt(var + EPS) * w2_ref[...]
    hln_ref[...] = hln.astype(jnp.bfloat16)
    lg_ref[...] = jnp.dot(
        hln,
        wr_ref[...],
        preferred_element_type=jnp.float32,
        precision=jax.lax.Precision.HIGHEST,
    )


def _post_attn(ao, wo16, x, ln2_w, wr, BS):
    S, D = x.shape
    E = wr.shape[1]
    row = pl.BlockSpec((BS, D), lambda i: (i, 0))
    return pl.pallas_call(
        _k3_body,
        grid=(S // BS,),
        in_specs=[
            row,
            pl.BlockSpec((D, D), lambda i: (0, 0)),
            row,
            pl.BlockSpec((1, D), lambda i: (0, 0)),
            pl.BlockSpec((D, E), lambda i: (0, 0)),
        ],
        out_specs=(row, row, pl.BlockSpec((BS, E), lambda i: (i, 0))),
        out_shape=(
            jax.ShapeDtypeStruct((S, D), jnp.float32),
            jax.ShapeDtypeStruct((S, D), jnp.bfloat16),
            jax.ShapeDtypeStruct((S, E), jnp.float32),
        ),
    )(ao, wo16, x, ln2_w.reshape(1, D), wr)


def _k4_body(lg_ref, st_ref, cidx_ref, gate_ref, *, S, E, C, CH):
    lg = lg_ref[...]
    m = jnp.max(lg, axis=-1, keepdims=True)
    ex = jnp.exp(lg - m)
    probs = ex / jnp.sum(ex, axis=-1, keepdims=True)

    iota_e = jax.lax.broadcasted_iota(jnp.int32, (S, E), 1)
    m1 = jnp.max(probs, axis=-1, keepdims=True)
    i1 = jnp.min(jnp.where(probs == m1, iota_e, E), axis=-1, keepdims=True)
    probs2 = jnp.where(iota_e == i1, -1.0, probs)
    m2 = jnp.max(probs2, axis=-1, keepdims=True)
    i2 = jnp.min(jnp.where(probs2 == m2, iota_e, E), axis=-1, keepdims=True)
    den = m1 + m2 + 1e-9
    g1 = m1 / den
    g2 = m2 / den

    oh0 = (iota_e == i1).astype(jnp.float32)
    oh1 = (iota_e == i2).astype(jnp.float32)
    A = jnp.concatenate([oh0, oh1], axis=0)

    tri = (
        jax.lax.broadcasted_iota(jnp.int32, (CH, CH), 0)
        > jax.lax.broadcasted_iota(jnp.int32, (CH, CH), 1)
    ).astype(jnp.float32)
    carry = jnp.zeros((1, E), jnp.float32)
    prefs = []
    for j in range(2 * S // CH):
        blk = jax.lax.slice(A, (j * CH, 0), ((j + 1) * CH, E))
        prefs.append(
            jnp.dot(
                tri,
                blk,
                preferred_element_type=jnp.float32,
                precision=jax.lax.Precision.HIGHEST,
            )
            + carry
        )
        carry = carry + jnp.sum(blk, axis=0, keepdims=True)
    pos = jnp.concatenate(prefs, axis=0)
    pos_sel = jnp.floor(
        jnp.sum(pos * A, axis=-1, keepdims=True) + 0.5
    )
    valid = (pos_sel < C).astype(jnp.float32)
    pos_c = jnp.clip(pos_sel, 0.0, C - 1.0)

    gate_all = jnp.concatenate([g1, g2], axis=0) * valid
    e_all = jnp.concatenate([i1, i2], axis=0).astype(jnp.float32)
    cidx_ref[...] = (e_all * C + pos_c).astype(jnp.int32)
    gate_ref[...] = gate_all

    iota_c = jax.lax.broadcasted_iota(jnp.int32, (2 * S, C), 1)
    P = (pos_c.astype(jnp.int32) == iota_c).astype(jnp.float32) * valid
    t_all = jax.lax.broadcasted_iota(jnp.int32, (2 * S, 1), 0).astype(
        jnp.float32
    )
    t_all = jnp.where(t_all < S, t_all, t_all - S)
    lhs = A * ((t_all + 1.0) * valid)
    st = jax.lax.dot_general(
        P,
        lhs,
        (((0,), (0,)), ((), ())),
        preferred_element_type=jnp.float32,
        precision=jax.lax.Precision.HIGHEST,
    )
    st_ref[...] = jnp.maximum((st + 0.5).astype(jnp.int32) - 1, 0)


def _routing(logits, C, CH):
    S, E = logits.shape
    body = functools.partial(_k4_body, S=S, E=E, C=C, CH=CH)
    return pl.pallas_call(
        body,
        grid=(1,),
        in_specs=[pl.BlockSpec((S, E), lambda i: (0, 0))],
        out_specs=(
            pl.BlockSpec((C, E), lambda i: (0, 0)),
            pl.BlockSpec((2 * S, 1), lambda i: (0, 0)),
            pl.BlockSpec((2 * S, 1), lambda i: (0, 0)),
        ),
        out_shape=(
            jax.ShapeDtypeStruct((C, E), jnp.int32),
            jax.ShapeDtypeStruct((2 * S, 1), jnp.int32),
            jax.ShapeDtypeStruct((2 * S, 1), jnp.float32),
        ),
    )(logits)


def _k5_body(st_ref, hln_ref, w1_ref, w3_ref, w2_ref, eo_ref, *, BC, S, E):
    e = pl.program_id(0)
    cb = pl.program_id(1)
    blk = st_ref[pl.ds(cb * BC, BC), :]
    lane = jax.lax.broadcasted_iota(jnp.int32, (BC, E), 1)
    ids = jnp.sum(jnp.where(lane == e, blk, 0), axis=1, keepdims=True)
    iota_s = jax.lax.broadcasted_iota(jnp.int32, (BC, S), 1)
    disp = (ids == iota_s).astype(jnp.bfloat16)
    buf = jnp.dot(
        disp, hln_ref[...], preferred_element_type=jnp.float32
    ).astype(jnp.bfloat16)
    h1 = jnp.dot(buf, w1_ref[0].astype(jnp.bfloat16),
                 preferred_element_type=jnp.float32)
    h3 = jnp.dot(buf, w3_ref[0].astype(jnp.bfloat16),
                 preferred_element_type=jnp.float32)
    hh = (h1 * jax.lax.logistic(h1) * h3).astype(jnp.bfloat16)
    eo_ref[...] = jnp.dot(hh, w2_ref[0].astype(jnp.bfloat16),
                          preferred_element_type=jnp.float32)


def _expert_ffn(st, hln16, w1, w3, w2, BC):
    C, E = st.shape
    S, D = hln16.shape
    F = w1.shape[2]
    nb = C // BC
    body = functools.partial(_k5_body, BC=BC, S=S, E=E)
    return pl.pallas_call(
        body,
        grid=(E, nb),
        in_specs=[
            pl.BlockSpec((C, E), lambda e, c: (0, 0)),
            pl.BlockSpec((S, D), lambda e, c: (0, 0)),
            pl.BlockSpec((1, D, F), lambda e, c: (e, 0, 0)),
            pl.BlockSpec((1, D, F), lambda e, c: (e, 0, 0)),
            pl.BlockSpec((1, F, D), lambda e, c: (e, 0, 0)),
        ],
        out_specs=pl.BlockSpec((BC, D), lambda e, c: (e * nb + c, 0)),
        out_shape=jax.ShapeDtypeStruct((E * C, D), jnp.float32),
    )(st, hln16, w1, w3, w2)


def _sc_combine_gather(eo, cidx3d):
    NSLOT, D = eo.shape
    KK, NW, TPW = cidx3d.shape
    S = NW * TPW
    mesh = plsc.VectorSubcoreMesh(
        core_axis_name="c", subcore_axis_name="s",
        num_cores=2, num_subcores=16,
    )

    @functools.partial(
        pl.kernel,
        mesh=mesh,
        out_type=jax.ShapeDtypeStruct((KK, S, D), jnp.float32),
        scratch_types=[
            pltpu.VMEM((KK, TPW), jnp.int32),
            pltpu.VMEM((TPW, D), jnp.float32),
            pltpu.SemaphoreType.DMA,
        ],
    )
    def k(eo_hbm, cidx_hbm, r_hbm, idx_v, rows_v, sem):
        cid = lax.axis_index("c")
        sid = lax.axis_index("s")
        wid = sid * 2 + cid
        for j in range(KK):
            pltpu.sync_copy(cidx_hbm.at[j, wid], idx_v.at[j])
        for j in range(KK):
            pltpu.async_copy(eo_hbm.at[idx_v.at[j]], rows_v, sem).wait()
            pltpu.sync_copy(rows_v, r_hbm.at[j, pl.ds(wid * TPW, TPW)])

    return k(eo, cidx3d)


def _k7_body(r_ref, gate_ref, h_ref, o_ref, *, BT, S):
    i = pl.program_id(0)
    g0 = gate_ref[pl.ds(i * BT, BT), :]
    g1 = gate_ref[pl.ds(S + i * BT, BT), :]
    o_ref[...] = h_ref[...] + g0 * r_ref[0] + g1 * r_ref[1]


def _combine_add(r, gate, h, BT):
    KK, S, D = r.shape
    body = functools.partial(_k7_body, BT=BT, S=S)
    return pl.pallas_call(
        body,
        grid=(S // BT,),
        in_specs=[
            pl.BlockSpec((KK, BT, D), lambda i: (0, i, 0)),
            pl.BlockSpec((2 * S, 1), lambda i: (0, 0)),
            pl.BlockSpec((BT, D), lambda i: (i, 0)),
        ],
        out_specs=pl.BlockSpec((BT, D), lambda i: (i, 0)),
        out_shape=jax.ShapeDtypeStruct((S, D), jnp.float32),
    )(r, gate, h)


def kernel(x, ln1_w, Wq, Wk, Wv, Wo, ln2_w, Wr, W1, W3, W2):
    B, S, D = x.shape
    E = Wr.shape[1]
    C = int(math.ceil(B * S * TOP_K / E * CAP_FACTOR))
    HD = D // N_HEADS
    xf = x.reshape(S, D)

    q, k, v = _qkv(xf, ln1_w, Wq, Wk, Wv, BS=256)
    ao = _attention(q, k, v, N_HEADS, BQ=min(512, S))
    h, hln, logits = _post_attn(ao, Wo, xf, ln2_w, Wr, BS=256)
    st, cidx, gate = _routing(logits, C=C, CH=min(512, 2 * S))
    eo = _expert_ffn(st, hln, W1, W3, W2, BC=C // 5)
    cidx3d = cidx.reshape(2, 32, S // 32)
    r = _sc_combine_gather(eo, cidx3d)
    out = _combine_add(r, gate, h, BT=256)
    return out.reshape(B, S, D)


# speedup vs baseline: 1.7294x; 1.0363x over previous
import functools
import math

import jax
import jax.numpy as jnp
from jax import lax
from jax.experimental import pallas as pl
from jax.experimental.pallas import tpu as pltpu
from jax.experimental.pallas import tpu_sc as plsc

EPS = 1e-5
N_HEADS = 16
TOP_K = 2
CAP_FACTOR = 1.25
NEG_INF = -1e30


def _k1_body(x_ref, w_ref, wq_ref, wk_ref, wv_ref, q_ref, k_ref, v_ref):
    x = x_ref[...]
    var = jnp.mean(x * x, axis=-1, keepdims=True)
    xln = (x * jax.lax.rsqrt(var + EPS) * w_ref[...]).astype(jnp.bfloat16)
    for wr, outr in ((wq_ref, q_ref), (wk_ref, k_ref), (wv_ref, v_ref)):
        outr[...] = jnp.dot(
            xln, wr[...].astype(jnp.bfloat16),
            preferred_element_type=jnp.float32,
        ).astype(jnp.bfloat16)


def _qkv(x, ln1_w, wq16, wk16, wv16, BS):
    S, D = x.shape
    row = pl.BlockSpec((BS, D), lambda i: (i, 0))
    full = pl.BlockSpec((D, D), lambda i: (0, 0))
    wspec = pl.BlockSpec((1, D), lambda i: (0, 0))
    out = jax.ShapeDtypeStruct((S, D), jnp.bfloat16)
    return pl.pallas_call(
        _k1_body,
        grid=(S // BS,),
        in_specs=[row, wspec, full, full, full],
        out_specs=(row, row, row),
        out_shape=(out, out, out),
    )(x, ln1_w.reshape(1, D), wq16, wk16, wv16)


def _k2_body(q_ref, k_ref, v_ref, o_ref, *, BQ, BK, HD, scale):
    qb = pl.program_id(1)
    q2 = q_ref[...]
    row = qb * BQ + jax.lax.broadcasted_iota(jnp.int32, (BQ, BK), 0)

    def step(j, carry):
        ma, la, acca, mb, lb, accb = carry
        kc = k_ref[pl.ds(j * BK, BK), :]
        vc = v_ref[pl.ds(j * BK, BK), :]
        col = j * BK + jax.lax.broadcasted_iota(jnp.int32, (BQ, BK), 1)
        mask = col <= row

        def upd(qh_, kh_, vh_, m, l, acc):
            s = jax.lax.dot_general(
                qh_, kh_, (((1,), (1,)), ((), ())),
                preferred_element_type=jnp.float32,
            ) * scale
            s = jnp.where(mask, s, NEG_INF)
            mn = jnp.maximum(m, jnp.max(s, axis=-1, keepdims=True))
            p = jnp.exp(s - mn)
            corr = jnp.exp(m - mn)
            l = l * corr + jnp.sum(p, axis=-1, keepdims=True)
            acc = acc * corr + jnp.dot(
                p.astype(jnp.bfloat16), vh_,
                preferred_element_type=jnp.float32,
            )
            return mn, l, acc

        ma, la, acca = upd(q2[:, :HD], kc[:, :HD], vc[:, :HD], ma, la, acca)
        mb, lb, accb = upd(q2[:, HD:], kc[:, HD:], vc[:, HD:], mb, lb, accb)
        return ma, la, acca, mb, lb, accb

    m0 = jnp.full((BQ, 1), NEG_INF, jnp.float32)
    l0 = jnp.zeros((BQ, 1), jnp.float32)
    a0 = jnp.zeros((BQ, HD), jnp.float32)
    ma, la, acca, mb, lb, accb = jax.lax.fori_loop(
        0, qb + 1, step, (m0, l0, a0, m0, l0, a0)
    )
    o_ref[...] = jnp.concatenate(
        [acca / la, accb / lb], axis=1
    ).astype(jnp.bfloat16)


def _attention(q, k, v, n_heads, BQ):
    S, D = q.shape
    HD = D // n_heads
    HP = n_heads // 2
    body = functools.partial(
        _k2_body, BQ=BQ, BK=BQ, HD=HD, scale=1.0 / math.sqrt(HD)
    )
    return pl.pallas_call(
        body,
        grid=(HP, S // BQ),
        in_specs=[
            pl.BlockSpec((BQ, 2 * HD), lambda hp, i: (i, hp)),
            pl.BlockSpec((S, 2 * HD), lambda hp, i: (0, hp)),
            pl.BlockSpec((S, 2 * HD), lambda hp, i: (0, hp)),
        ],
        out_specs=pl.BlockSpec((BQ, 2 * HD), lambda hp, i: (i, hp)),
        out_shape=jax.ShapeDtypeStruct((S, D), jnp.bfloat16),
    )(q, k, v)


def _k3_body(ao_ref, wo_ref, x_ref, w2_ref, wr_ref, h_ref, hln_ref, lg_ref):
    att = jnp.dot(
        ao_ref[...], wo_ref[...].astype(jnp.bfloat16),
        preferred_element_type=jnp.float32,
    )
    h = att + x_ref[...]
    h_ref[...] = h
    var = jnp.mean(h * h, axis=-1, keepdims=True)
    hln = h * jax.lax.rsqrt(var + EPS) * w2_ref[...]
    hln_ref[...] = hln.astype(jnp.bfloat16)
    lg_ref[...] = jnp.dot(
        hln,
        wr_ref[...],
        preferred_element_type=jnp.float32,
        precision=jax.lax.Precision.HIGHEST,
    )


def _post_attn(ao, wo16, x, ln2_w, wr, BS):
    S, D = x.shape
    E = wr.shape[1]
    row = pl.BlockSpec((BS, D), lambda i: (i, 0))
    return pl.pallas_call(
        _k3_body,
        grid=(S // BS,),
        in_specs=[
            row,
            pl.BlockSpec((D, D), lambda i: (0, 0)),
            row,
            pl.BlockSpec((1, D), lambda i: (0, 0)),
            pl.BlockSpec((D, E), lambda i: (0, 0)),
        ],
        out_specs=(row, row, pl.BlockSpec((BS, E), lambda i: (i, 0))),
        out_shape=(
            jax.ShapeDtypeStruct((S, D), jnp.float32),
            jax.ShapeDtypeStruct((S, D), jnp.bfloat16),
            jax.ShapeDtypeStruct((S, E), jnp.float32),
        ),
    )(ao, wo16, x, ln2_w.reshape(1, D), wr)


def _k4_body(lg_ref, st_ref, cidx_ref, gate_ref, *, S, E, C, CH):
    lg = lg_ref[...]
    m = jnp.max(lg, axis=-1, keepdims=True)
    ex = jnp.exp(lg - m)
    probs = ex / jnp.sum(ex, axis=-1, keepdims=True)

    iota_e = jax.lax.broadcasted_iota(jnp.int32, (S, E), 1)
    m1 = jnp.max(probs, axis=-1, keepdims=True)
    i1 = jnp.min(jnp.where(probs == m1, iota_e, E), axis=-1, keepdims=True)
    probs2 = jnp.where(iota_e == i1, -1.0, probs)
    m2 = jnp.max(probs2, axis=-1, keepdims=True)
    i2 = jnp.min(jnp.where(probs2 == m2, iota_e, E), axis=-1, keepdims=True)
    den = m1 + m2 + 1e-9
    g1 = m1 / den
    g2 = m2 / den

    oh0 = (iota_e == i1).astype(jnp.float32)
    oh1 = (iota_e == i2).astype(jnp.float32)
    A = jnp.concatenate([oh0, oh1], axis=0)

    tri = (
        jax.lax.broadcasted_iota(jnp.int32, (CH, CH), 0)
        > jax.lax.broadcasted_iota(jnp.int32, (CH, CH), 1)
    ).astype(jnp.float32)
    carry = jnp.zeros((1, E), jnp.float32)
    prefs = []
    for j in range(2 * S // CH):
        blk = jax.lax.slice(A, (j * CH, 0), ((j + 1) * CH, E))
        prefs.append(
            jnp.dot(
                tri,
                blk,
                preferred_element_type=jnp.float32,
                precision=jax.lax.Precision.HIGHEST,
            )
            + carry
        )
        carry = carry + jnp.sum(blk, axis=0, keepdims=True)
    pos = jnp.concatenate(prefs, axis=0)
    pos_sel = jnp.floor(
        jnp.sum(pos * A, axis=-1, keepdims=True) + 0.5
    )
    valid = (pos_sel < C).astype(jnp.float32)
    pos_c = jnp.clip(pos_sel, 0.0, C - 1.0)

    gate_all = jnp.concatenate([g1, g2], axis=0) * valid
    e_all = jnp.concatenate([i1, i2], axis=0).astype(jnp.float32)
    cidx_ref[...] = (e_all * C + pos_c).astype(jnp.int32)
    gate_ref[...] = gate_all

    iota_c = jax.lax.broadcasted_iota(jnp.int32, (2 * S, C), 1)
    P = (pos_c.astype(jnp.int32) == iota_c).astype(jnp.float32) * valid
    t_all = jax.lax.broadcasted_iota(jnp.int32, (2 * S, 1), 0).astype(
        jnp.float32
    )
    t_all = jnp.where(t_all < S, t_all, t_all - S)
    lhs = A * ((t_all + 1.0) * valid)
    st = jax.lax.dot_general(
        P,
        lhs,
        (((0,), (0,)), ((), ())),
        preferred_element_type=jnp.float32,
        precision=jax.lax.Precision.HIGHEST,
    )
    st_ref[...] = jnp.maximum((st + 0.5).astype(jnp.int32) - 1, 0)


def _routing(logits, C, CH):
    S, E = logits.shape
    body = functools.partial(_k4_body, S=S, E=E, C=C, CH=CH)
    return pl.pallas_call(
        body,
        grid=(1,),
        in_specs=[pl.BlockSpec((S, E), lambda i: (0, 0))],
        out_specs=(
            pl.BlockSpec((C, E), lambda i: (0, 0)),
            pl.BlockSpec((2 * S, 1), lambda i: (0, 0)),
            pl.BlockSpec((2 * S, 1), lambda i: (0, 0)),
        ),
        out_shape=(
            jax.ShapeDtypeStruct((C, E), jnp.int32),
            jax.ShapeDtypeStruct((2 * S, 1), jnp.int32),
            jax.ShapeDtypeStruct((2 * S, 1), jnp.float32),
        ),
    )(logits)


def _k5_body(st_ref, hln_ref, w1_ref, w3_ref, w2_ref, eo_ref, *, BC, S, E):
    e = pl.program_id(0)
    cb = pl.program_id(1)
    blk = st_ref[pl.ds(cb * BC, BC), :]
    lane = jax.lax.broadcasted_iota(jnp.int32, (BC, E), 1)
    ids = jnp.sum(jnp.where(lane == e, blk, 0), axis=1, keepdims=True)
    iota_s = jax.lax.broadcasted_iota(jnp.int32, (BC, S), 1)
    disp = (ids == iota_s).astype(jnp.bfloat16)
    buf = jnp.dot(
        disp, hln_ref[...], preferred_element_type=jnp.float32
    ).astype(jnp.bfloat16)
    h1 = jnp.dot(buf, w1_ref[0].astype(jnp.bfloat16),
                 preferred_element_type=jnp.float32)
    h3 = jnp.dot(buf, w3_ref[0].astype(jnp.bfloat16),
                 preferred_element_type=jnp.float32)
    hh = (h1 * jax.lax.logistic(h1) * h3).astype(jnp.bfloat16)
    eo_ref[...] = jnp.dot(hh, w2_ref[0].astype(jnp.bfloat16),
                          preferred_element_type=jnp.float32
                          ).astype(jnp.bfloat16)


def _expert_ffn(st, hln16, w1, w3, w2, BC):
    C, E = st.shape
    S, D = hln16.shape
    F = w1.shape[2]
    nb = C // BC
    body = functools.partial(_k5_body, BC=BC, S=S, E=E)
    return pl.pallas_call(
        body,
        grid=(E, nb),
        in_specs=[
            pl.BlockSpec((C, E), lambda e, c: (0, 0)),
            pl.BlockSpec((S, D), lambda e, c: (0, 0)),
            pl.BlockSpec((1, D, F), lambda e, c: (e, 0, 0)),
            pl.BlockSpec((1, D, F), lambda e, c: (e, 0, 0)),
            pl.BlockSpec((1, F, D), lambda e, c: (e, 0, 0)),
        ],
        out_specs=pl.BlockSpec((BC, D), lambda e, c: (e * nb + c, 0)),
        out_shape=jax.ShapeDtypeStruct((E * C, D), jnp.bfloat16),
    )(st, hln16, w1, w3, w2)


def _k6_body(cidx_ref, gate_ref, eo_ref, h_ref, o_ref, *, BT, S, NS):
    i = pl.program_id(0)
    c0 = cidx_ref[pl.ds(i * BT, BT), :]
    c1 = cidx_ref[pl.ds(S + i * BT, BT), :]
    g0 = gate_ref[pl.ds(i * BT, BT), :]
    g1 = gate_ref[pl.ds(S + i * BT, BT), :]
    iota_ns = jax.lax.broadcasted_iota(jnp.int32, (BT, NS), 1)
    G = (
        (c0 == iota_ns).astype(jnp.float32) * g0
        + (c1 == iota_ns).astype(jnp.float32) * g1
    ).astype(jnp.bfloat16)
    y = jnp.dot(G, eo_ref[...], preferred_element_type=jnp.float32)
    o_ref[...] = h_ref[...] + y


def _combine(cidx, gate, eo, h, BT):
    NS, D = eo.shape
    S = h.shape[0]
    body = functools.partial(_k6_body, BT=BT, S=S, NS=NS)
    return pl.pallas_call(
        body,
        grid=(S // BT,),
        in_specs=[
            pl.BlockSpec((2 * S, 1), lambda i: (0, 0)),
            pl.BlockSpec((2 * S, 1), lambda i: (0, 0)),
            pl.BlockSpec((NS, D), lambda i: (0, 0)),
            pl.BlockSpec((BT, D), lambda i: (i, 0)),
        ],
        out_specs=pl.BlockSpec((BT, D), lambda i: (i, 0)),
        out_shape=jax.ShapeDtypeStruct((S, D), jnp.float32),
    )(cidx, gate, eo, h)


def kernel(x, ln1_w, Wq, Wk, Wv, Wo, ln2_w, Wr, W1, W3, W2):
    B, S, D = x.shape
    E = Wr.shape[1]
    C = int(math.ceil(B * S * TOP_K / E * CAP_FACTOR))
    HD = D // N_HEADS
    xf = x.reshape(S, D)

    q, k, v = _qkv(xf, ln1_w, Wq, Wk, Wv, BS=256)
    ao = _attention(q, k, v, N_HEADS, BQ=min(512, S))
    h, hln, logits = _post_attn(ao, Wo, xf, ln2_w, Wr, BS=256)
    st, cidx, gate = _routing(logits, C=C, CH=min(512, 2 * S))
    eo = _expert_ffn(st, hln, W1, W3, W2, BC=C // 5)
    out = _combine(cidx, gate, eo, h, BT=256)
    return out.reshape(B, S, D)


# speedup vs baseline: 1.7968x; 1.0390x over previous
import functools
import math

import jax
import jax.numpy as jnp
from jax import lax
from jax.experimental import pallas as pl
from jax.experimental.pallas import tpu as pltpu
from jax.experimental.pallas import tpu_sc as plsc

EPS = 1e-5
N_HEADS = 16
TOP_K = 2
CAP_FACTOR = 1.25
NEG_INF = -1e30


def _k1_body(x_ref, w_ref, wq_ref, wk_ref, wv_ref, q_ref, k_ref, v_ref):
    x = x_ref[...]
    var = jnp.mean(x * x, axis=-1, keepdims=True)
    xln = (x * jax.lax.rsqrt(var + EPS) * w_ref[...]).astype(jnp.bfloat16)
    for wr, outr in ((wq_ref, q_ref), (wk_ref, k_ref), (wv_ref, v_ref)):
        outr[...] = jnp.dot(
            xln, wr[...].astype(jnp.bfloat16),
            preferred_element_type=jnp.float32,
        ).astype(jnp.bfloat16)


def _qkv(x, ln1_w, wq16, wk16, wv16, BS):
    S, D = x.shape
    row = pl.BlockSpec((BS, D), lambda i: (i, 0))
    full = pl.BlockSpec((D, D), lambda i: (0, 0))
    wspec = pl.BlockSpec((1, D), lambda i: (0, 0))
    out = jax.ShapeDtypeStruct((S, D), jnp.bfloat16)
    return pl.pallas_call(
        _k1_body,
        grid=(S // BS,),
        in_specs=[row, wspec, full, full, full],
        out_specs=(row, row, row),
        out_shape=(out, out, out),
    )(x, ln1_w.reshape(1, D), wq16, wk16, wv16)


def _k2_body(q_ref, k_ref, v_ref, o_ref, *, BQ, BK, HD, scale):
    qb = pl.program_id(1)
    q2 = q_ref[...]
    row = qb * BQ + jax.lax.broadcasted_iota(jnp.int32, (BQ, BK), 0)

    def step(j, carry):
        ma, la, acca, mb, lb, accb = carry
        kc = k_ref[pl.ds(j * BK, BK), :]
        vc = v_ref[pl.ds(j * BK, BK), :]
        col = j * BK + jax.lax.broadcasted_iota(jnp.int32, (BQ, BK), 1)
        mask = col <= row

        def upd(qh_, kh_, vh_, m, l, acc):
            s = jax.lax.dot_general(
                qh_, kh_, (((1,), (1,)), ((), ())),
                preferred_element_type=jnp.float32,
            ) * scale
            s = jnp.where(mask, s, NEG_INF)
            mn = jnp.maximum(m, jnp.max(s, axis=-1, keepdims=True))
            p = jnp.exp(s - mn)
            corr = jnp.exp(m - mn)
            l = l * corr + jnp.sum(p, axis=-1, keepdims=True)
            acc = acc * corr + jnp.dot(
                p.astype(jnp.bfloat16), vh_,
                preferred_element_type=jnp.float32,
            )
            return mn, l, acc

        ma, la, acca = upd(q2[:, :HD], kc[:, :HD], vc[:, :HD], ma, la, acca)
        mb, lb, accb = upd(q2[:, HD:], kc[:, HD:], vc[:, HD:], mb, lb, accb)
        return ma, la, acca, mb, lb, accb

    m0 = jnp.full((BQ, 1), NEG_INF, jnp.float32)
    l0 = jnp.zeros((BQ, 1), jnp.float32)
    a0 = jnp.zeros((BQ, HD), jnp.float32)
    ma, la, acca, mb, lb, accb = jax.lax.fori_loop(
        0, qb + 1, step, (m0, l0, a0, m0, l0, a0)
    )
    o_ref[...] = jnp.concatenate(
        [acca / la, accb / lb], axis=1
    ).astype(jnp.bfloat16)


def _attention(q, k, v, n_heads, BQ):
    S, D = q.shape
    HD = D // n_heads
    HP = n_heads // 2
    body = functools.partial(
        _k2_body, BQ=BQ, BK=BQ, HD=HD, scale=1.0 / math.sqrt(HD)
    )
    return pl.pallas_call(
        body,
        grid=(HP, S // BQ),
        in_specs=[
            pl.BlockSpec((BQ, 2 * HD), lambda hp, i: (i, hp)),
            pl.BlockSpec((S, 2 * HD), lambda hp, i: (0, hp)),
            pl.BlockSpec((S, 2 * HD), lambda hp, i: (0, hp)),
        ],
        out_specs=pl.BlockSpec((BQ, 2 * HD), lambda hp, i: (i, hp)),
        out_shape=jax.ShapeDtypeStruct((S, D), jnp.bfloat16),
    )(q, k, v)


def _k3_body(ao_ref, wo_ref, x_ref, w2_ref, wr_ref, h_ref, hln_ref, lg_ref):
    att = jnp.dot(
        ao_ref[...], wo_ref[...].astype(jnp.bfloat16),
        preferred_element_type=jnp.float32,
    )
    h = att + x_ref[...]
    h_ref[...] = h
    var = jnp.mean(h * h, axis=-1, keepdims=True)
    hln = h * jax.lax.rsqrt(var + EPS) * w2_ref[...]
    hln_ref[...] = hln.astype(jnp.bfloat16)
    lg_ref[...] = jnp.dot(
        hln,
        wr_ref[...],
        preferred_element_type=jnp.float32,
        precision=jax.lax.Precision.HIGHEST,
    )


def _post_attn(ao, wo16, x, ln2_w, wr, BS):
    S, D = x.shape
    E = wr.shape[1]
    row = pl.BlockSpec((BS, D), lambda i: (i, 0))
    return pl.pallas_call(
        _k3_body,
        grid=(S // BS,),
        in_specs=[
            row,
            pl.BlockSpec((D, D), lambda i: (0, 0)),
            row,
            pl.BlockSpec((1, D), lambda i: (0, 0)),
            pl.BlockSpec((D, E), lambda i: (0, 0)),
        ],
        out_specs=(row, row, pl.BlockSpec((BS, E), lambda i: (i, 0))),
        out_shape=(
            jax.ShapeDtypeStruct((S, D), jnp.float32),
            jax.ShapeDtypeStruct((S, D), jnp.bfloat16),
            jax.ShapeDtypeStruct((S, E), jnp.float32),
        ),
    )(ao, wo16, x, ln2_w.reshape(1, D), wr)


def _k4_body(lg_ref, st_ref, cidx_ref, gate_ref, *, S, E, C, CH):
    lg = lg_ref[...]
    m = jnp.max(lg, axis=-1, keepdims=True)
    ex = jnp.exp(lg - m)
    probs = ex / jnp.sum(ex, axis=-1, keepdims=True)

    iota_e = jax.lax.broadcasted_iota(jnp.int32, (S, E), 1)
    m1 = jnp.max(probs, axis=-1, keepdims=True)
    i1 = jnp.min(jnp.where(probs == m1, iota_e, E), axis=-1, keepdims=True)
    probs2 = jnp.where(iota_e == i1, -1.0, probs)
    m2 = jnp.max(probs2, axis=-1, keepdims=True)
    i2 = jnp.min(jnp.where(probs2 == m2, iota_e, E), axis=-1, keepdims=True)
    den = m1 + m2 + 1e-9
    g1 = m1 / den
    g2 = m2 / den

    oh0 = (iota_e == i1).astype(jnp.float32)
    oh1 = (iota_e == i2).astype(jnp.float32)
    A = jnp.concatenate([oh0, oh1], axis=0)

    tri = (
        jax.lax.broadcasted_iota(jnp.int32, (CH, CH), 0)
        > jax.lax.broadcasted_iota(jnp.int32, (CH, CH), 1)
    ).astype(jnp.float32)
    carry = jnp.zeros((1, E), jnp.float32)
    prefs = []
    for j in range(2 * S // CH):
        blk = jax.lax.slice(A, (j * CH, 0), ((j + 1) * CH, E))
        prefs.append(
            jnp.dot(
                tri,
                blk,
                preferred_element_type=jnp.float32,
                precision=jax.lax.Precision.HIGHEST,
            )
            + carry
        )
        carry = carry + jnp.sum(blk, axis=0, keepdims=True)
    pos = jnp.concatenate(prefs, axis=0)
    pos_sel = jnp.floor(
        jnp.sum(pos * A, axis=-1, keepdims=True) + 0.5
    )
    valid = (pos_sel < C).astype(jnp.float32)
    pos_c = jnp.clip(pos_sel, 0.0, C - 1.0)

    gate_all = jnp.concatenate([g1, g2], axis=0) * valid
    e_all = jnp.concatenate([i1, i2], axis=0).astype(jnp.float32)
    cidx_ref[...] = (e_all * C + pos_c).astype(jnp.int32)
    gate_ref[...] = gate_all

    iota_c = jax.lax.broadcasted_iota(jnp.int32, (2 * S, C), 1)
    P = (pos_c.astype(jnp.int32) == iota_c).astype(jnp.float32) * valid
    t_all = jax.lax.broadcasted_iota(jnp.int32, (2 * S, 1), 0).astype(
        jnp.float32
    )
    t_all = jnp.where(t_all < S, t_all, t_all - S)
    lhs = A * ((t_all + 1.0) * valid)
    st = jax.lax.dot_general(
        P,
        lhs,
        (((0,), (0,)), ((), ())),
        preferred_element_type=jnp.float32,
        precision=jax.lax.Precision.HIGHEST,
    )
    st_ref[...] = jnp.maximum((st + 0.5).astype(jnp.int32) - 1, 0)


def _routing(logits, C, CH):
    S, E = logits.shape
    body = functools.partial(_k4_body, S=S, E=E, C=C, CH=CH)
    return pl.pallas_call(
        body,
        grid=(1,),
        in_specs=[pl.BlockSpec((S, E), lambda i: (0, 0))],
        out_specs=(
            pl.BlockSpec((C, E), lambda i: (0, 0)),
            pl.BlockSpec((2 * S, 1), lambda i: (0, 0)),
            pl.BlockSpec((2 * S, 1), lambda i: (0, 0)),
        ),
        out_shape=(
            jax.ShapeDtypeStruct((C, E), jnp.int32),
            jax.ShapeDtypeStruct((2 * S, 1), jnp.int32),
            jax.ShapeDtypeStruct((2 * S, 1), jnp.float32),
        ),
    )(logits)


def _k5_body(st_ref, hln_ref, w1_ref, w3_ref, w2_ref, eo_ref, *, BC, S, E):
    e = pl.program_id(0)
    cb = pl.program_id(1)
    blk = st_ref[pl.ds(cb * BC, BC), :]
    lane = jax.lax.broadcasted_iota(jnp.int32, (BC, E), 1)
    ids = jnp.sum(jnp.where(lane == e, blk, 0), axis=1, keepdims=True)
    iota_s = jax.lax.broadcasted_iota(jnp.int32, (BC, S), 1)
    disp = (ids == iota_s).astype(jnp.bfloat16)
    buf = jnp.dot(
        disp, hln_ref[...], preferred_element_type=jnp.float32
    ).astype(jnp.bfloat16)
    h1 = jnp.dot(buf, w1_ref[0].astype(jnp.bfloat16),
                 preferred_element_type=jnp.float32)
    h3 = jnp.dot(buf, w3_ref[0].astype(jnp.bfloat16),
                 preferred_element_type=jnp.float32)
    hh = (h1 * jax.lax.logistic(h1) * h3).astype(jnp.bfloat16)
    eo_ref[...] = jnp.dot(hh, w2_ref[0].astype(jnp.bfloat16),
                          preferred_element_type=jnp.float32)


def _expert_ffn(st, hln16, w1, w3, w2, BC):
    C, E = st.shape
    S, D = hln16.shape
    F = w1.shape[2]
    nb = C // BC
    body = functools.partial(_k5_body, BC=BC, S=S, E=E)
    return pl.pallas_call(
        body,
        grid=(E, nb),
        in_specs=[
            pl.BlockSpec((C, E), lambda e, c: (0, 0)),
            pl.BlockSpec((S, D), lambda e, c: (0, 0)),
            pl.BlockSpec((1, D, F), lambda e, c: (e, 0, 0)),
            pl.BlockSpec((1, D, F), lambda e, c: (e, 0, 0)),
            pl.BlockSpec((1, F, D), lambda e, c: (e, 0, 0)),
        ],
        out_specs=pl.BlockSpec((BC, D), lambda e, c: (e * nb + c, 0)),
        out_shape=jax.ShapeDtypeStruct((E * C, D), jnp.float32),
    )(st, hln16, w1, w3, w2)


def _sc_combine_gather(eo, cidx3d):
    NSLOT, D = eo.shape
    KK, NW, TPW = cidx3d.shape
    S = NW * TPW
    mesh = plsc.VectorSubcoreMesh(
        core_axis_name="c", subcore_axis_name="s",
        num_cores=2, num_subcores=16,
    )

    @functools.partial(
        pl.kernel,
        mesh=mesh,
        out_type=jax.ShapeDtypeStruct((KK, S, D), jnp.float32),
        scratch_types=[
            pltpu.VMEM((KK, TPW), jnp.int32),
            pltpu.VMEM((TPW, D), jnp.float32),
            pltpu.SemaphoreType.DMA,
        ],
    )
    def k(eo_hbm, cidx_hbm, r_hbm, idx_v, rows_v, sem):
        cid = lax.axis_index("c")
        sid = lax.axis_index("s")
        wid = sid * 2 + cid
        for j in range(KK):
            pltpu.sync_copy(cidx_hbm.at[j, wid], idx_v.at[j])
        for j in range(KK):
            pltpu.async_copy(eo_hbm.at[idx_v.at[j]], rows_v, sem).wait()
            pltpu.sync_copy(rows_v, r_hbm.at[j, pl.ds(wid * TPW, TPW)])

    return k(eo, cidx3d)


def _k7_body(r_ref, gate_ref, h_ref, o_ref, *, BT, S):
    i = pl.program_id(0)
    g0 = gate_ref[pl.ds(i * BT, BT), :]
    g1 = gate_ref[pl.ds(S + i * BT, BT), :]
    o_ref[...] = h_ref[...] + g0 * r_ref[0] + g1 * r_ref[1]


def _combine_add(r, gate, h, BT):
    KK, S, D = r.shape
    body = functools.partial(_k7_body, BT=BT, S=S)
    return pl.pallas_call(
        body,
        grid=(S // BT,),
        in_specs=[
            pl.BlockSpec((KK, BT, D), lambda i: (0, i, 0)),
            pl.BlockSpec((2 * S, 1), lambda i: (0, 0)),
            pl.BlockSpec((BT, D), lambda i: (i, 0)),
        ],
        out_specs=pl.BlockSpec((BT, D), lambda i: (i, 0)),
        out_shape=jax.ShapeDtypeStruct((S, D), jnp.float32),
    )(r, gate, h)


def kernel(x, ln1_w, Wq, Wk, Wv, Wo, ln2_w, Wr, W1, W3, W2):
    B, S, D = x.shape
    E = Wr.shape[1]
    C = int(math.ceil(B * S * TOP_K / E * CAP_FACTOR))
    HD = D // N_HEADS
    xf = x.reshape(S, D)

    q, k, v = _qkv(xf, ln1_w, Wq, Wk, Wv, BS=256)
    ao = _attention(q, k, v, N_HEADS, BQ=min(512, S))
    h, hln, logits = _post_attn(ao, Wo, xf, ln2_w, Wr, BS=256)
    st, cidx, gate = _routing(logits, C=C, CH=min(512, 2 * S))
    eo = _expert_ffn(st, hln, W1, W3, W2, BC=C // 2)
    cidx3d = cidx.reshape(2, 32, S // 32)
    r = _sc_combine_gather(eo, cidx3d)
    out = _combine_add(r, gate, h, BT=256)
    return out.reshape(B, S, D)
